# baseline (speedup 1.0000x reference)
import functools

import jax
import jax.numpy as jnp
from jax import lax
from jax.experimental import pallas as pl
from jax.experimental.pallas import tpu as pltpu
from jax.experimental.pallas import tpu_sc as plsc

N = 10000
E = 320000
D = 128
NC = 2
NS = 16
NW = NC * NS
G = 16
EPAD = 327680
NGRP_TOTAL = EPAD // G
NGRP = NGRP_TOTAL // NW
NBUF = 8
CGRP = 320
ROWS_W = N // NS
INIT_CH = 125

_mesh = plsc.VectorSubcoreMesh(core_axis_name="c", subcore_axis_name="s")


def _deg_body(dst_hbm, ew_hbm, out_hbm, dstb, ewb, deg, _sem):
    cid = lax.axis_index("c")
    sid = lax.axis_index("s")
    w = cid * NS + sid
    pltpu.sync_copy(dst_hbm.at[pl.ds(w * NGRP, NGRP)], dstb)
    pltpu.sync_copy(ew_hbm.at[pl.ds(w * NGRP, NGRP)], ewb)

    zero = jnp.zeros((16,), jnp.float32)

    @pl.loop(0, N // 16)
    def _(i):
        deg[pl.ds(i * 16, 16)] = zero

    @pl.loop(0, NGRP)
    def _(g):
        d16 = dstb[g]
        w16 = ewb[g]
        plsc.addupdate_scatter(deg, [d16], w16)

    pltpu.sync_copy(deg, out_hbm.at[w])


_sc_params = pltpu.CompilerParams(use_tc_tiling_on_sc=False,
                                  needs_layout_passes=False)

_deg_call = pl.kernel(
    _deg_body,
    out_type=jax.ShapeDtypeStruct((NW, N), jnp.float32),
    mesh=_mesh,
    compiler_params=_sc_params,
    scratch_types=[
        pltpu.VMEM((NGRP, G), jnp.int32),
        pltpu.VMEM((NGRP, G), jnp.float32),
        pltpu.VMEM((N,), jnp.float32),
        pltpu.SemaphoreType.DMA,
    ],
)


def _spmm_body(tab_hbm, tabf_hbm, src_hbm, dst_hbm, ew_hbm, out_hbm,
               srcb, dstb, ewb, rin, rout, accum, gsem, ssem):
    cid = lax.axis_index("c")
    sid = lax.axis_index("s")
    w = cid * NS + sid

    row0 = sid * ROWS_W
    pltpu.sync_copy(tab_hbm.at[pl.ds(row0, ROWS_W)], accum.at[pl.ds(row0, ROWS_W)])
    plsc.subcore_barrier()

    for chunk in range(NGRP // CGRP):
        base = w * NGRP + chunk * CGRP
        pltpu.sync_copy(src_hbm.at[pl.ds(base, CGRP)], srcb)
        pltpu.sync_copy(dst_hbm.at[pl.ds(base, CGRP)], dstb)
        pltpu.sync_copy(ew_hbm.at[pl.ds(base, CGRP)], ewb)

        for b in range(NBUF):
            pltpu.async_copy(tabf_hbm.at[srcb.at[b]], rin.at[b], gsem.at[b])

        @pl.loop(0, CGRP // NBUF)
        def _(i):
            g0 = i * NBUF
            for b in range(NBUF):
                g = g0 + b

                @pl.when(i > 0)
                def _():
                    pltpu.make_async_copy(
                        rout.at[b], accum.at[dstb.at[g - NBUF]], ssem.at[b]
                    ).wait()

                pltpu.make_async_copy(
                    tabf_hbm.at[srcb.at[g]], rin.at[b], gsem.at[b]
                ).wait()

                ew16 = ewb[g]
                for r in range(G):
                    s = ew16[r]
                    for j in range(D // 32):
                        v = rin[b, r, pl.ds(j * 32, 32)]
                        lo, hi = plsc.unpack(v, format=plsc.PackFormat.INTERLEAVED)
                        rout[b, r, pl.ds(j * 32, 16)] = lo * s
                        rout[b, r, pl.ds(j * 32 + 16, 16)] = hi * s

                @pl.when(g + NBUF < CGRP)
                def _():
                    pltpu.async_copy(
                        tabf_hbm.at[srcb.at[g + NBUF]], rin.at[b], gsem.at[b]
                    )

                pltpu.async_copy(
                    rout.at[b], accum.at[dstb.at[g]], ssem.at[b], add=True
                )

        for b in range(NBUF):
            g = CGRP - NBUF + b
            pltpu.make_async_copy(
                rout.at[b], accum.at[dstb.at[g]], ssem.at[b]
            ).wait()
    plsc.subcore_barrier()

    pltpu.sync_copy(accum.at[pl.ds(row0, ROWS_W)],
                    out_hbm.at[cid, pl.ds(row0, ROWS_W)])


_spmm_call = pl.kernel(
    _spmm_body,
    out_type=jax.ShapeDtypeStruct((NC, N, D), jnp.float32),
    mesh=_mesh,
    compiler_params=_sc_params,
    scratch_types=[
        pltpu.VMEM((CGRP, G), jnp.int32),
        pltpu.VMEM((CGRP, G), jnp.int32),
        pltpu.VMEM((CGRP, G), jnp.float32),
        pltpu.VMEM((NBUF, G, D), jnp.bfloat16),
        pltpu.VMEM((NBUF, G, D), jnp.float32),
        pltpu.VMEM_SHARED((N, D), jnp.float32),
        pltpu.SemaphoreType.DMA((NBUF,)),
        pltpu.SemaphoreType.DMA((NBUF,)),
    ],
)


_RB = 2000


def _tc1_body(degs_ref, x_ref, dis_ref, xp_ref):
    deg = jnp.sum(degs_ref[...], axis=0) + 1.0
    dis = lax.rsqrt(deg)[:, None]
    dis_ref[...] = dis
    xp_ref[...] = x_ref[...] * dis


def _tc1(degs, x):
    return pl.pallas_call(
        _tc1_body,
        out_shape=[
            jax.ShapeDtypeStruct((N, 1), jnp.float32),
            jax.ShapeDtypeStruct((N, D), jnp.float32),
        ],
    )(degs, x)


def _tc2_body(p_ref, xp_ref, dis_ref, w1_ref, b1_ref, w2_ref, h2p_ref):
    dis = dis_ref[...]
    ax = (p_ref[0] + p_ref[1] - xp_ref[...]) * dis
    h1 = jnp.dot(ax, w1_ref[...], preferred_element_type=jnp.float32)
    h1 = jnp.maximum(h1 + b1_ref[...], 0.0)
    h2 = jnp.dot(h1, w2_ref[...], preferred_element_type=jnp.float32)
    h2p_ref[...] = h2 * dis


def _tc2(p, xp, dis, W1, b1, W2):
    dh = W1.shape[1]
    return pl.pallas_call(
        _tc2_body,
        grid=(N // _RB,),
        in_specs=[
            pl.BlockSpec((NC, _RB, D), lambda i: (0, i, 0)),
            pl.BlockSpec((_RB, D), lambda i: (i, 0)),
            pl.BlockSpec((_RB, 1), lambda i: (i, 0)),
            pl.BlockSpec((D, dh), lambda i: (0, 0)),
            pl.BlockSpec((1, dh), lambda i: (0, 0)),
            pl.BlockSpec((dh, D), lambda i: (0, 0)),
        ],
        out_specs=pl.BlockSpec((_RB, D), lambda i: (i, 0)),
        out_shape=jax.ShapeDtypeStruct((N, D), jnp.float32),
    )(p, xp, dis, W1, b1, W2)


def _tc3_body(q_ref, h2p_ref, dis_ref, b2_ref, out_ref):
    agg = (q_ref[0] + q_ref[1] - h2p_ref[...]) * dis_ref[...]
    out_ref[...] = agg + b2_ref[...]


def _tc3(q, h2p, dis, b2):
    return pl.pallas_call(
        _tc3_body,
        grid=(N // _RB,),
        in_specs=[
            pl.BlockSpec((NC, _RB, D), lambda i: (0, i, 0)),
            pl.BlockSpec((_RB, D), lambda i: (i, 0)),
            pl.BlockSpec((_RB, 1), lambda i: (i, 0)),
            pl.BlockSpec((1, D), lambda i: (0, 0)),
        ],
        out_specs=pl.BlockSpec((_RB, D), lambda i: (i, 0)),
        out_shape=jax.ShapeDtypeStruct((N, D), jnp.float32),
    )(q, h2p, dis, b2)


def kernel(x, edge_index, edge_weight, W1, b1, W2, b2):
    npad = EPAD - E
    ipad = jnp.arange(npad, dtype=edge_index.dtype) % N
    src = jnp.concatenate([edge_index[0], ipad]).reshape(NGRP_TOTAL, G)
    dst = jnp.concatenate([edge_index[1], ipad]).reshape(NGRP_TOTAL, G)
    ew = jnp.concatenate(
        [edge_weight, jnp.zeros((npad,), edge_weight.dtype)]
    ).reshape(NGRP_TOTAL, G)

    degs = _deg_call(dst, ew)
    dis, xp = _tc1(degs, x)
    p = _spmm_call(xp, _permbf(xp), src, dst, ew)
    h2p = _tc2(p, xp, dis, W1, b1.reshape(1, -1), W2)
    q = _spmm_call(h2p, _permbf(h2p), src, dst, ew)
    return _tc3(q, h2p, dis, b2.reshape(1, -1))


def _permbf(t):
    n = t.shape[0]
    return (t.reshape(n, D // 32, 2, 16).swapaxes(2, 3)
             .reshape(n, D).astype(jnp.bfloat16))


# speedup vs baseline: 1.0458x; 1.0458x over previous
import functools

import jax
import jax.numpy as jnp
from jax import lax
from jax.experimental import pallas as pl
from jax.experimental.pallas import tpu as pltpu
from jax.experimental.pallas import tpu_sc as plsc

N = 10000
E = 320000
D = 128
NC = 2
NS = 16
NW = NC * NS
G = 16
EPAD = 327680
NGRP_TOTAL = EPAD // G
NGRP = NGRP_TOTAL // NW
NBUF = 8
CGRP = 320
ROWS_W = N // NS
INIT_CH = 125

_mesh = plsc.VectorSubcoreMesh(core_axis_name="c", subcore_axis_name="s")


def _deg_body(dst_hbm, ew_hbm, out_hbm, dstb, ewb, deg, _sem):
    cid = lax.axis_index("c")
    sid = lax.axis_index("s")
    w = cid * NS + sid
    pltpu.sync_copy(dst_hbm.at[pl.ds(w * NGRP, NGRP)], dstb)
    pltpu.sync_copy(ew_hbm.at[pl.ds(w * NGRP, NGRP)], ewb)

    zero = jnp.zeros((16,), jnp.float32)

    @pl.loop(0, N // 16)
    def _(i):
        deg[pl.ds(i * 16, 16)] = zero

    @pl.loop(0, NGRP)
    def _(g):
        d16 = dstb[g]
        w16 = ewb[g]
        plsc.addupdate_scatter(deg, [d16], w16)

    pltpu.sync_copy(deg, out_hbm.at[w])


_sc_params = pltpu.CompilerParams(use_tc_tiling_on_sc=False,
                                  needs_layout_passes=False)

_deg_call = pl.kernel(
    _deg_body,
    out_type=jax.ShapeDtypeStruct((NW, N), jnp.float32),
    mesh=_mesh,
    compiler_params=_sc_params,
    scratch_types=[
        pltpu.VMEM((NGRP, G), jnp.int32),
        pltpu.VMEM((NGRP, G), jnp.float32),
        pltpu.VMEM((N,), jnp.float32),
        pltpu.SemaphoreType.DMA,
    ],
)


def _spmm_body(tab_hbm, src_hbm, dst_hbm, ew_hbm, out_hbm,
               srcb, dstb, ewb, rin, rout, accum, gsem, ssem):
    cid = lax.axis_index("c")
    sid = lax.axis_index("s")
    w = cid * NS + sid

    row0 = sid * ROWS_W
    pltpu.sync_copy(tab_hbm.at[pl.ds(row0, ROWS_W)], accum.at[pl.ds(row0, ROWS_W)])
    plsc.subcore_barrier()

    for chunk in range(NGRP // CGRP):
        base = w * NGRP + chunk * CGRP
        pltpu.sync_copy(src_hbm.at[pl.ds(base, CGRP)], srcb)
        pltpu.sync_copy(dst_hbm.at[pl.ds(base, CGRP)], dstb)
        pltpu.sync_copy(ew_hbm.at[pl.ds(base, CGRP)], ewb)

        for b in range(NBUF):
            pltpu.async_copy(tab_hbm.at[srcb.at[b]], rin.at[b], gsem.at[b])

        @pl.loop(0, CGRP // NBUF)
        def _(i):
            g0 = i * NBUF
            for b in range(NBUF):
                g = g0 + b

                @pl.when(i > 0)
                def _():
                    pltpu.make_async_copy(
                        rout.at[b], accum.at[dstb.at[g - NBUF]], ssem.at[b]
                    ).wait()

                pltpu.make_async_copy(
                    tab_hbm.at[srcb.at[g]], rin.at[b], gsem.at[b]
                ).wait()

                ew16 = ewb[g]
                for r in range(G):
                    s = ew16[r]
                    for c in range(D // 16):
                        sl = pl.ds(c * 16, 16)
                        rout[b, r, sl] = rin[b, r, sl] * s

                @pl.when(g + NBUF < CGRP)
                def _():
                    pltpu.async_copy(
                        tab_hbm.at[srcb.at[g + NBUF]], rin.at[b], gsem.at[b]
                    )

                pltpu.async_copy(
                    rout.at[b], accum.at[dstb.at[g]], ssem.at[b], add=True
                )

        for b in range(NBUF):
            g = CGRP - NBUF + b
            pltpu.make_async_copy(
                rout.at[b], accum.at[dstb.at[g]], ssem.at[b]
            ).wait()
    plsc.subcore_barrier()

    pltpu.sync_copy(accum.at[pl.ds(row0, ROWS_W)],
                    out_hbm.at[cid, pl.ds(row0, ROWS_W)])


_spmm_call = pl.kernel(
    _spmm_body,
    out_type=jax.ShapeDtypeStruct((NC, N, D), jnp.float32),
    mesh=_mesh,
    compiler_params=_sc_params,
    scratch_types=[
        pltpu.VMEM((CGRP, G), jnp.int32),
        pltpu.VMEM((CGRP, G), jnp.int32),
        pltpu.VMEM((CGRP, G), jnp.float32),
        pltpu.VMEM((NBUF, G, D), jnp.float32),
        pltpu.VMEM((NBUF, G, D), jnp.float32),
        pltpu.VMEM_SHARED((N, D), jnp.float32),
        pltpu.SemaphoreType.DMA((NBUF,)),
        pltpu.SemaphoreType.DMA((NBUF,)),
    ],
)


_RB = 2000


def _tc1_body(degs_ref, x_ref, dis_ref, xp_ref):
    deg = jnp.sum(degs_ref[...], axis=0) + 1.0
    dis = lax.rsqrt(deg)[:, None]
    dis_ref[...] = dis
    xp_ref[...] = x_ref[...] * dis


def _tc1(degs, x):
    return pl.pallas_call(
        _tc1_body,
        out_shape=[
            jax.ShapeDtypeStruct((N, 1), jnp.float32),
            jax.ShapeDtypeStruct((N, D), jnp.float32),
        ],
    )(degs, x)


def _tc2_body(p_ref, xp_ref, dis_ref, w1_ref, b1_ref, w2_ref, h2p_ref):
    dis = dis_ref[...]
    ax = (p_ref[0] + p_ref[1] - xp_ref[...]) * dis
    h1 = jnp.dot(ax, w1_ref[...], preferred_element_type=jnp.float32)
    h1 = jnp.maximum(h1 + b1_ref[...], 0.0)
    h2 = jnp.dot(h1, w2_ref[...], preferred_element_type=jnp.float32)
    h2p_ref[...] = h2 * dis


def _tc2(p, xp, dis, W1, b1, W2):
    dh = W1.shape[1]
    return pl.pallas_call(
        _tc2_body,
        grid=(N // _RB,),
        in_specs=[
            pl.BlockSpec((NC, _RB, D), lambda i: (0, i, 0)),
            pl.BlockSpec((_RB, D), lambda i: (i, 0)),
            pl.BlockSpec((_RB, 1), lambda i: (i, 0)),
            pl.BlockSpec((D, dh), lambda i: (0, 0)),
            pl.BlockSpec((1, dh), lambda i: (0, 0)),
            pl.BlockSpec((dh, D), lambda i: (0, 0)),
        ],
        out_specs=pl.BlockSpec((_RB, D), lambda i: (i, 0)),
        out_shape=jax.ShapeDtypeStruct((N, D), jnp.float32),
    )(p, xp, dis, W1, b1, W2)


def _tc3_body(q_ref, h2p_ref, dis_ref, b2_ref, out_ref):
    agg = (q_ref[0] + q_ref[1] - h2p_ref[...]) * dis_ref[...]
    out_ref[...] = agg + b2_ref[...]


def _tc3(q, h2p, dis, b2):
    return pl.pallas_call(
        _tc3_body,
        grid=(N // _RB,),
        in_specs=[
            pl.BlockSpec((NC, _RB, D), lambda i: (0, i, 0)),
            pl.BlockSpec((_RB, D), lambda i: (i, 0)),
            pl.BlockSpec((_RB, 1), lambda i: (i, 0)),
            pl.BlockSpec((1, D), lambda i: (0, 0)),
        ],
        out_specs=pl.BlockSpec((_RB, D), lambda i: (i, 0)),
        out_shape=jax.ShapeDtypeStruct((N, D), jnp.float32),
    )(q, h2p, dis, b2)


def kernel(x, edge_index, edge_weight, W1, b1, W2, b2):
    npad = EPAD - E
    ipad = jnp.arange(npad, dtype=edge_index.dtype) % N
    src = jnp.concatenate([edge_index[0], ipad]).reshape(NGRP_TOTAL, G)
    dst = jnp.concatenate([edge_index[1], ipad]).reshape(NGRP_TOTAL, G)
    ew = jnp.concatenate(
        [edge_weight, jnp.zeros((npad,), edge_weight.dtype)]
    ).reshape(NGRP_TOTAL, G)

    degs = _deg_call(dst, ew)
    dis, xp = _tc1(degs, x)
    p = _spmm_call(xp, src, dst, ew)
    h2p = _tc2(p, xp, dis, W1, b1.reshape(1, -1), W2)
    q = _spmm_call(h2p, src, dst, ew)
    return _tc3(q, h2p, dis, b2.reshape(1, -1))
